# baseline (speedup 1.0000x reference)
import jax
import jax.numpy as jnp
from jax.experimental import pallas as pl

CHUNK = 32
MICRO_STEPS = 3
HALO = 64

_DNT = (((1,), (1,)), ((), ()))


def _bf(a):
    return a.astype(jnp.bfloat16)


def _dgt(a, b):
    return jax.lax.dot_general(a, b, _DNT,
                               preferred_element_type=jnp.float32)


def _prep(W1_ref, Wp_ref, bp_ref, b1_ref, Wq_ref, b1q_ref):
    D = Wp_ref.shape[0]
    w1c = W1_ref[:, 2 * D:]
    Wq_ref[...] = _bf(jnp.dot(w1c, Wp_ref[...],
                              preferred_element_type=jnp.float32))
    b1q_ref[...] = b1_ref[...] + _dgt(bp_ref[...], w1c)


def _block(x_ref, xh_ref, Win_ref, bin_ref, cw_ref, cb_ref, Wq_ref,
           bhead_ref, W1_ref, Whead_ref, b1q_ref,
           W2_ref, b2_ref, Wout_ref, bout_ref,
           out_ref, sal_ref, unc_ref, halt_ref):
    T = x_ref.shape[1]
    D = Win_ref.shape[0]
    K = cw_ref.shape[0]
    W = T + HALO
    C = W // CHUNK

    i = pl.program_id(1)
    row = jax.lax.broadcasted_iota(jnp.int32, (W, 1), 0)
    mask = jnp.where((row >= HALO) | (i > 0), 1.0, 0.0).astype(jnp.float32)

    xw = jnp.concatenate([xh_ref[0, 0], x_ref[0]], axis=0)
    h = _dgt(_bf(xw), Win_ref[...])
    h = (h + bin_ref[...]) * mask

    for step in range(MICRO_STEPS):
        h_b = _bf(h)

        padded = jnp.concatenate(
            [jnp.zeros((K - 1, D), jnp.bfloat16), h_b], axis=0)
        mixed = cb_ref[...]
        for j in range(K):
            mixed = mixed + padded[j:j + W, :] * cw_ref[j:j + 1, :]

        ph = jnp.mean(h_b.reshape(C, CHUNK, D), axis=1)
        pb = _dgt(ph, Wq_ref[...]) + b1q_ref[...]
        pbr = jnp.broadcast_to(pb[:, None, :], (C, CHUNK, D)).reshape(W, D)

        pre = (_dgt(h_b, W1_ref[:, :D])
               + _dgt(mixed, W1_ref[:, D:2 * D])
               + pbr)
        t = jnp.tanh(pre)
        delta = _dgt(_bf(t), W2_ref[...]) + b2_ref[...]

        hlog = _dgt(h_b, Whead_ref[...])
        halt = jax.nn.sigmoid(hlog[:, 2:3] + bhead_ref[0:1, 2:3])

        if step == MICRO_STEPS - 1:
            sl = hlog[:, 0:1] + bhead_ref[0:1, 0:1]
            ul = hlog[:, 1:2] + bhead_ref[0:1, 1:2]
            sal_ref[0, 0] = jnp.sum(jax.nn.sigmoid(sl)[HALO:], keepdims=True)
            unc_ref[0, 0] = jnp.sum(jax.nn.sigmoid(ul)[HALO:], keepdims=True)
            halt_ref[0, 0] = jnp.sum(halt[HALO:], keepdims=True)

        h = (h + delta * halt) * mask

    out = x_ref[0] + _dgt(_bf(h[HALO:]), Wout_ref[...]) + bout_ref[...]
    out_ref[0] = out


@jax.jit
def kernel(x, W_in, b_in, conv_w, conv_b, W_p, b_p, W_head, b_head,
           W1, b1, W2, b2, W_out, b_out):
    B, S, MD = x.shape
    D = W_in.shape[0]
    T = 1024 if S % 1024 == 0 else S
    NB = S // T

    Win_b = _bf(W_in)
    cw_t = _bf(conv_w.T)
    cb_row = _bf(conv_b.reshape(1, -1))
    W1_b = _bf(W1)
    Whead_b = _bf(W_head)
    W2_b = _bf(W2)
    Wout_b = _bf(W_out)

    def row(v):
        return v.reshape(1, -1)

    Wq_b, b1q = pl.pallas_call(
        _prep,
        in_specs=[pl.BlockSpec(w.shape, (lambda n: lambda: (0,) * n)(w.ndim))
                  for w in (W1, W_p, row(b_p), row(b1))],
        out_specs=[pl.BlockSpec((D, D), lambda: (0, 0)),
                   pl.BlockSpec((1, D), lambda: (0, 0))],
        out_shape=[jax.ShapeDtypeStruct((D, D), jnp.bfloat16),
                   jax.ShapeDtypeStruct((1, D), jnp.float32)],
    )(W1, W_p, row(b_p), row(b1))

    halos = [jnp.zeros((B, 1, HALO, MD), x.dtype)]
    for i in range(1, NB):
        halos.append(x[:, None, i * T - HALO:i * T, :])
    xh = jnp.concatenate(halos, axis=1)

    full = lambda a: pl.BlockSpec(a.shape, lambda b, i: (0,) * a.ndim)
    operands = [Win_b, row(b_in), cw_t, cb_row, Wq_b,
                row(b_head), W1_b, Whead_b, b1q,
                W2_b, row(b2), Wout_b, row(b_out)]

    sum_spec = pl.BlockSpec((1, 1, 1, 1), lambda b, i: (b, i, 0, 0))
    sum_shape = jax.ShapeDtypeStruct((B, NB, 1, 1), jnp.float32)
    out, sal, unc, hal = pl.pallas_call(
        _block,
        grid=(B, NB),
        in_specs=[pl.BlockSpec((1, T, MD), lambda b, i: (b, i, 0)),
                  pl.BlockSpec((1, 1, HALO, MD), lambda b, i: (b, i, 0, 0))]
                 + [full(a) for a in operands],
        out_specs=[pl.BlockSpec((1, T, MD), lambda b, i: (b, i, 0)),
                   sum_spec, sum_spec, sum_spec],
        out_shape=[jax.ShapeDtypeStruct((B, S, MD), jnp.float32),
                   sum_shape, sum_shape, sum_shape],
    )(x, xh, *operands)

    denom = jnp.float32(B * S)
    return (out, jnp.sum(sal) / denom, jnp.sum(unc) / denom,
            jnp.sum(hal) / denom)


# speedup vs baseline: 1.1626x; 1.1626x over previous
import jax
import jax.numpy as jnp
from jax.experimental import pallas as pl

CHUNK = 32
MICRO_STEPS = 3
HALO = 64
HPAD = 128


def _bf(a):
    return a.astype(jnp.bfloat16)


def _prep(Win_ref, Wp_ref, W1_ref, Whead_ref, W2_ref, Wout_ref,
          bp_ref, b1_ref,
          Wint_ref, Wq_ref, W1aW_ref, W1bt_ref, W2t_ref, Woutt_ref, b1q_ref):
    D = Wp_ref.shape[0]
    w1c = W1_ref[:, 2 * D:]
    Wint_ref[...] = jnp.transpose(_bf(Win_ref[...]))
    w = jnp.dot(w1c, Wp_ref[...], preferred_element_type=jnp.float32)
    Wq_ref[...] = jnp.transpose(_bf(w))
    W1aW_ref[...] = jnp.concatenate(
        [jnp.transpose(_bf(W1_ref[:, :D])),
         jnp.transpose(_bf(Whead_ref[...])),
         jnp.zeros((D, HPAD - Whead_ref.shape[0]), jnp.bfloat16)], axis=1)
    W1bt_ref[...] = jnp.transpose(_bf(W1_ref[:, D:2 * D]))
    W2t_ref[...] = jnp.transpose(_bf(W2_ref[...]))
    Woutt_ref[...] = jnp.transpose(_bf(Wout_ref[...]))
    b1q_ref[...] = b1_ref[...] + jax.lax.dot_general(
        bp_ref[...], w1c, (((1,), (1,)), ((), ())),
        preferred_element_type=jnp.float32)


def _block(x_ref, xh_ref, Win_ref, bin_ref, cw_ref, cb_ref, Wq_ref,
           bhead_ref, W1aW_ref, W1b_ref, b1q_ref,
           W2_ref, b2_ref, Wout_ref, bout_ref,
           out_ref, sal_ref, unc_ref, halt_ref):
    T = x_ref.shape[1]
    D = Win_ref.shape[1]
    K = cw_ref.shape[0]
    W = T + HALO
    C = W // CHUNK

    i = pl.program_id(1)
    row = jax.lax.broadcasted_iota(jnp.int32, (W, 1), 0)
    mask = jnp.where((row >= HALO) | (i > 0), 1.0, 0.0).astype(jnp.float32)

    xw = jnp.concatenate([xh_ref[0, 0], x_ref[0]], axis=0)
    h = jnp.dot(_bf(xw), Win_ref[...], preferred_element_type=jnp.float32)
    h = (h + bin_ref[...]) * mask

    for step in range(MICRO_STEPS):
        h_b = _bf(h)

        padded = jnp.concatenate(
            [jnp.zeros((K - 1, D), jnp.bfloat16), h_b], axis=0)
        mixed = cb_ref[...]
        for j in range(K):
            mixed = mixed + padded[j:j + W, :] * cw_ref[j:j + 1, :]

        ph = jnp.mean(h_b.reshape(C, CHUNK, D), axis=1)
        pb = jnp.dot(ph, Wq_ref[...],
                     preferred_element_type=jnp.float32) + b1q_ref[...]
        pbr = jnp.broadcast_to(pb[:, None, :], (C, CHUNK, D)).reshape(W, D)

        full = jnp.dot(h_b, W1aW_ref[...],
                       preferred_element_type=jnp.float32)
        pre = (full[:, :D]
               + jnp.dot(mixed, W1b_ref[...],
                         preferred_element_type=jnp.float32)
               + pbr)
        t = jnp.tanh(pre)
        delta = jnp.dot(_bf(t), W2_ref[...],
                        preferred_element_type=jnp.float32) + b2_ref[...]

        halt = jax.nn.sigmoid(full[:, D + 2:D + 3] + bhead_ref[0:1, 2:3])

        if step == MICRO_STEPS - 1:
            sl = full[:, D:D + 1] + bhead_ref[0:1, 0:1]
            ul = full[:, D + 1:D + 2] + bhead_ref[0:1, 1:2]
            sal_ref[0, 0] = jnp.sum(jax.nn.sigmoid(sl)[HALO:], keepdims=True)
            unc_ref[0, 0] = jnp.sum(jax.nn.sigmoid(ul)[HALO:], keepdims=True)
            halt_ref[0, 0] = jnp.sum(halt[HALO:], keepdims=True)

        h = (h + delta * halt) * mask

    out = x_ref[0] + jnp.dot(_bf(h[HALO:]), Wout_ref[...],
                             preferred_element_type=jnp.float32) + bout_ref[...]
    out_ref[0] = out


@jax.jit
def kernel(x, W_in, b_in, conv_w, conv_b, W_p, b_p, W_head, b_head,
           W1, b1, W2, b2, W_out, b_out):
    B, S, MD = x.shape
    D = W_in.shape[0]
    T = 1024 if S % 1024 == 0 else S
    NB = S // T

    cw_t = _bf(conv_w.T)
    cb_row = _bf(conv_b.reshape(1, -1))

    def row(v):
        return v.reshape(1, -1)

    prep_in = (W_in, W_p, W1, W_head, W2, W_out, row(b_p), row(b1))
    Win_t, Wq_t, W1aW_t, W1b_t, W2_t, Wout_t, b1q = pl.pallas_call(
        _prep,
        in_specs=[pl.BlockSpec(w.shape, (lambda n: lambda: (0,) * n)(w.ndim))
                  for w in prep_in],
        out_specs=[pl.BlockSpec(s, (lambda n: lambda: (0,) * n)(2))
                   for s in ((MD, D), (D, D), (D, D + HPAD), (D, D),
                             (D, D), (D, MD), (1, D))],
        out_shape=[jax.ShapeDtypeStruct((MD, D), jnp.bfloat16),
                   jax.ShapeDtypeStruct((D, D), jnp.bfloat16),
                   jax.ShapeDtypeStruct((D, D + HPAD), jnp.bfloat16),
                   jax.ShapeDtypeStruct((D, D), jnp.bfloat16),
                   jax.ShapeDtypeStruct((D, D), jnp.bfloat16),
                   jax.ShapeDtypeStruct((D, MD), jnp.bfloat16),
                   jax.ShapeDtypeStruct((1, D), jnp.float32)],
    )(*prep_in)

    halos = [jnp.zeros((B, 1, HALO, MD), x.dtype)]
    for i in range(1, NB):
        halos.append(x[:, None, i * T - HALO:i * T, :])
    xh = jnp.concatenate(halos, axis=1)

    full = lambda a: pl.BlockSpec(a.shape, lambda b, i: (0,) * a.ndim)
    operands = [Win_t, row(b_in), cw_t, cb_row, Wq_t,
                row(b_head), W1aW_t, W1b_t, b1q,
                W2_t, row(b2), Wout_t, row(b_out)]

    sum_spec = pl.BlockSpec((1, 1, 1, 1), lambda b, i: (b, i, 0, 0))
    sum_shape = jax.ShapeDtypeStruct((B, NB, 1, 1), jnp.float32)
    out, sal, unc, hal = pl.pallas_call(
        _block,
        grid=(B, NB),
        in_specs=[pl.BlockSpec((1, T, MD), lambda b, i: (b, i, 0)),
                  pl.BlockSpec((1, 1, HALO, MD), lambda b, i: (b, i, 0, 0))]
                 + [full(a) for a in operands],
        out_specs=[pl.BlockSpec((1, T, MD), lambda b, i: (b, i, 0)),
                   sum_spec, sum_spec, sum_spec],
        out_shape=[jax.ShapeDtypeStruct((B, S, MD), jnp.float32),
                   sum_shape, sum_shape, sum_shape],
    )(x, xh, *operands)

    denom = jnp.float32(B * S)
    return (out, jnp.sum(sal) / denom, jnp.sum(unc) / denom,
            jnp.sum(hal) / denom)


# speedup vs baseline: 1.2791x; 1.1002x over previous
import jax
import jax.numpy as jnp
from jax.experimental import pallas as pl
from jax.experimental.pallas import tpu as pltpu

CHUNK = 32
MICRO_STEPS = 3
HPAD = 128
TAIL = 8


def _bf(a):
    return a.astype(jnp.bfloat16)


def _prep(Win_ref, Wp_ref, W1_ref, Whead_ref, W2_ref, Wout_ref,
          bp_ref, b1_ref,
          Wint_ref, Wq_ref, W1aW_ref, W1bt_ref, W2t_ref, Woutt_ref, b1q_ref):
    D = Wp_ref.shape[0]
    w1c = W1_ref[:, 2 * D:]
    Wint_ref[...] = jnp.transpose(_bf(Win_ref[...]))
    w = jnp.dot(w1c, Wp_ref[...], preferred_element_type=jnp.float32)
    Wq_ref[...] = jnp.transpose(_bf(w))
    W1aW_ref[...] = jnp.concatenate(
        [jnp.transpose(_bf(W1_ref[:, :D])),
         jnp.transpose(_bf(Whead_ref[...])),
         jnp.zeros((D, HPAD - Whead_ref.shape[0]), jnp.bfloat16)], axis=1)
    W1bt_ref[...] = jnp.transpose(_bf(W1_ref[:, D:2 * D]))
    W2t_ref[...] = jnp.transpose(_bf(W2_ref[...]))
    Woutt_ref[...] = jnp.transpose(_bf(Wout_ref[...]))
    b1q_ref[...] = b1_ref[...] + jax.lax.dot_general(
        bp_ref[...], w1c, (((1,), (1,)), ((), ())),
        preferred_element_type=jnp.float32)


def _block(x_ref, Win_ref, bin_ref, cw_ref, cb_ref, Wq_ref,
           bhead_ref, W1aW_ref, W1b_ref, b1q_ref,
           W2_ref, b2_ref, Wout_ref, bout_ref,
           out_ref, sal_ref, unc_ref, halt_ref, tail_ref):
    T = x_ref.shape[1]
    D = Win_ref.shape[1]
    K = cw_ref.shape[0]
    C = T // CHUNK

    i = pl.program_id(1)
    first = i == 0

    x = x_ref[0]
    h = jnp.dot(_bf(x), Win_ref[...], preferred_element_type=jnp.float32)
    h = h + bin_ref[...]

    for step in range(MICRO_STEPS):
        h_b = _bf(h)

        hist = jnp.where(first, jnp.zeros((TAIL, D), jnp.bfloat16),
                         tail_ref[step])
        tail_ref[step] = h_b[T - TAIL:]
        padded = jnp.concatenate([hist[TAIL - (K - 1):], h_b], axis=0)
        mixed = cb_ref[...]
        for j in range(K):
            mixed = mixed + padded[j:j + T, :] * cw_ref[j:j + 1, :]

        ph = jnp.mean(h_b.reshape(C, CHUNK, D), axis=1)
        pb = jnp.dot(ph, Wq_ref[...],
                     preferred_element_type=jnp.float32) + b1q_ref[...]
        pbr = jnp.broadcast_to(pb[:, None, :], (C, CHUNK, D)).reshape(T, D)

        full = jnp.dot(h_b, W1aW_ref[...],
                       preferred_element_type=jnp.float32)
        pre = (full[:, :D]
               + jnp.dot(mixed, W1b_ref[...],
                         preferred_element_type=jnp.float32)
               + pbr)
        t = jnp.tanh(pre)
        delta = jnp.dot(_bf(t), W2_ref[...],
                        preferred_element_type=jnp.float32) + b2_ref[...]

        halt = jax.nn.sigmoid(full[:, D + 2:D + 3] + bhead_ref[0:1, 2:3])

        if step == MICRO_STEPS - 1:
            sl = full[:, D:D + 1] + bhead_ref[0:1, 0:1]
            ul = full[:, D + 1:D + 2] + bhead_ref[0:1, 1:2]
            sal_ref[0, 0] = jnp.sum(jax.nn.sigmoid(sl), keepdims=True)
            unc_ref[0, 0] = jnp.sum(jax.nn.sigmoid(ul), keepdims=True)
            halt_ref[0, 0] = jnp.sum(halt, keepdims=True)

        h = h + delta * halt

    out_ref[0] = x + jnp.dot(_bf(h), Wout_ref[...],
                             preferred_element_type=jnp.float32) + bout_ref[...]


@jax.jit
def kernel(x, W_in, b_in, conv_w, conv_b, W_p, b_p, W_head, b_head,
           W1, b1, W2, b2, W_out, b_out):
    B, S, MD = x.shape
    D = W_in.shape[0]
    T = 1024 if S % 1024 == 0 else S
    NB = S // T

    cw_t = _bf(conv_w.T)
    cb_row = _bf(conv_b.reshape(1, -1))

    def row(v):
        return v.reshape(1, -1)

    prep_in = (W_in, W_p, W1, W_head, W2, W_out, row(b_p), row(b1))
    Win_t, Wq_t, W1aW_t, W1b_t, W2_t, Wout_t, b1q = pl.pallas_call(
        _prep,
        in_specs=[pl.BlockSpec(w.shape, (lambda n: lambda: (0,) * n)(w.ndim))
                  for w in prep_in],
        out_specs=[pl.BlockSpec(s, (lambda n: lambda: (0,) * n)(2))
                   for s in ((MD, D), (D, D), (D, D + HPAD), (D, D),
                             (D, D), (D, MD), (1, D))],
        out_shape=[jax.ShapeDtypeStruct((MD, D), jnp.bfloat16),
                   jax.ShapeDtypeStruct((D, D), jnp.bfloat16),
                   jax.ShapeDtypeStruct((D, D + HPAD), jnp.bfloat16),
                   jax.ShapeDtypeStruct((D, D), jnp.bfloat16),
                   jax.ShapeDtypeStruct((D, D), jnp.bfloat16),
                   jax.ShapeDtypeStruct((D, MD), jnp.bfloat16),
                   jax.ShapeDtypeStruct((1, D), jnp.float32)],
    )(*prep_in)

    full = lambda a: pl.BlockSpec(a.shape, lambda b, i: (0,) * a.ndim)
    operands = [Win_t, row(b_in), cw_t, cb_row, Wq_t,
                row(b_head), W1aW_t, W1b_t, b1q,
                W2_t, row(b2), Wout_t, row(b_out)]

    sum_spec = pl.BlockSpec((1, 1, 1, 1), lambda b, i: (b, i, 0, 0))
    sum_shape = jax.ShapeDtypeStruct((B, NB, 1, 1), jnp.float32)
    out, sal, unc, hal = pl.pallas_call(
        _block,
        grid=(B, NB),
        in_specs=[pl.BlockSpec((1, T, MD), lambda b, i: (b, i, 0))]
                 + [full(a) for a in operands],
        out_specs=[pl.BlockSpec((1, T, MD), lambda b, i: (b, i, 0)),
                   sum_spec, sum_spec, sum_spec],
        out_shape=[jax.ShapeDtypeStruct((B, S, MD), jnp.float32),
                   sum_shape, sum_shape, sum_shape],
        scratch_shapes=[pltpu.VMEM((MICRO_STEPS, TAIL, D), jnp.bfloat16)],
    )(x, *operands)

    denom = jnp.float32(B * S)
    return (out, jnp.sum(sal) / denom, jnp.sum(unc) / denom,
            jnp.sum(hal) / denom)
